# Initial kernel scaffold; baseline (speedup 1.0000x reference)
import jax
import jax.numpy as jnp
from jax.experimental import pallas as pl


def kernel(x, node_matrix, W, att_src, att_dst, bias):
    raise NotImplementedError("write your pallas kernel here")


# speedup vs baseline: 6279.9393x; 6279.9393x over previous
import functools

import jax
import jax.numpy as jnp
from jax.experimental import pallas as pl
from jax.experimental.pallas import tpu as pltpu


def _gat_block_kernel(x_ref, xb_ref, w_ref, as_ref, ad_ref, mask_ref,
                      bias_ref, out_ref, *, n, t_steps, heads, dim, bj):
    j = pl.program_id(0)
    col0 = j * bj
    m = mask_ref[...]
    rows = jax.lax.broadcasted_iota(jnp.int32, (n, bj), 0)
    cols = jax.lax.broadcasted_iota(jnp.int32, (n, bj), 1) + col0
    valid = (m != 0) | (rows == cols)
    w = w_ref[...]
    b = bias_ref[...]
    inv_h = jnp.float32(1.0 / heads)
    for t in range(t_steps):
        ht = jnp.dot(x_ref[:, t, :], w,
                     preferred_element_type=jnp.float32)
        htb = jnp.dot(xb_ref[:, t, :], w,
                      preferred_element_type=jnp.float32)
        a_src = jnp.dot(ht, as_ref[...],
                        preferred_element_type=jnp.float32)
        a_dst = jax.lax.dot_general(
            ad_ref[...], htb, (((1,), (1,)), ((), ())),
            preferred_element_type=jnp.float32)
        acc = None
        for hh in range(heads):
            s = a_src[:, hh:hh + 1] + a_dst[hh:hh + 1, :]
            s = jnp.where(s >= 0, s, 0.2 * s)
            s = jnp.where(valid, s, -1e30)
            mx = jnp.max(s, axis=0, keepdims=True)
            ex = jnp.exp(s - mx)
            den = jnp.sum(ex, axis=0, keepdims=True) + 1e-16
            alpha = ex / den
            o = jax.lax.dot_general(
                alpha, ht[:, hh * dim:(hh + 1) * dim],
                (((0,), (0,)), ((), ())),
                preferred_element_type=jnp.float32)
            acc = o if acc is None else acc + o
        out_ref[:, t * dim:(t + 1) * dim] = acc * inv_h + b


def kernel(x, node_matrix, W, att_src, att_dst, bias):
    n, t_steps, in_dim = x.shape
    heads, dim = att_src.shape[1], att_src.shape[2]
    hc = heads * dim
    bj = 256
    nj = n // bj

    eye = jnp.eye(heads, dtype=jnp.float32)
    as_bd = (att_src.reshape(heads, dim)[:, :, None]
             * eye[:, None, :]).reshape(hc, heads)
    ad_bd = (att_dst.reshape(heads, dim)[:, None, :]
             * eye[:, :, None]).reshape(heads, hc)
    bias2 = bias.reshape(1, dim).astype(jnp.float32)

    body = functools.partial(_gat_block_kernel, n=n, t_steps=t_steps,
                             heads=heads, dim=dim, bj=bj)
    out = pl.pallas_call(
        body,
        grid=(nj,),
        in_specs=[
            pl.BlockSpec((n, t_steps, in_dim), lambda j: (0, 0, 0)),
            pl.BlockSpec((bj, t_steps, in_dim), lambda j: (j, 0, 0)),
            pl.BlockSpec((in_dim, hc), lambda j: (0, 0)),
            pl.BlockSpec((hc, heads), lambda j: (0, 0)),
            pl.BlockSpec((heads, hc), lambda j: (0, 0)),
            pl.BlockSpec((n, bj), lambda j: (0, j)),
            pl.BlockSpec((1, dim), lambda j: (0, 0)),
        ],
        out_specs=pl.BlockSpec((bj, t_steps * dim), lambda j: (j, 0)),
        out_shape=jax.ShapeDtypeStruct((n, t_steps * dim), jnp.float32),
        compiler_params=pltpu.CompilerParams(
            dimension_semantics=("arbitrary",)),
    )(x.astype(jnp.float32), x.astype(jnp.float32), W, as_bd, ad_bd,
      node_matrix, bias2)
    return out.reshape(n, t_steps, dim)


# speedup vs baseline: 7307.8371x; 1.1637x over previous
import functools

import jax
import jax.numpy as jnp
from jax.experimental import pallas as pl
from jax.experimental.pallas import tpu as pltpu


def _gat_block_kernel(x_ref, xb_ref, w_ref, as_ref, ad_ref, mask_ref,
                      bias_ref, out_ref, *, n, t_steps, heads, dim, bj):
    j = pl.program_id(0)
    col0 = j * bj
    m = mask_ref[...]
    rows = jax.lax.broadcasted_iota(jnp.int32, (n, bj), 0)
    cols = jax.lax.broadcasted_iota(jnp.int32, (n, bj), 1) + col0
    valid = (m != 0) | (rows == cols)
    w = w_ref[...]
    b = bias_ref[...]
    inv_h = jnp.float32(1.0 / heads)
    log2e = jnp.float32(1.4426950408889634)
    ones = jnp.ones((n, 1), dtype=jnp.float32)
    for t in range(t_steps):
        ht = jnp.dot(x_ref[:, t, :], w,
                     preferred_element_type=jnp.float32)
        htb = jnp.dot(xb_ref[:, t, :], w,
                      preferred_element_type=jnp.float32)
        a_src = jnp.dot(ht, as_ref[...],
                        preferred_element_type=jnp.float32) * log2e
        a_dst = jax.lax.dot_general(
            ad_ref[...], htb, (((1,), (1,)), ((), ())),
            preferred_element_type=jnp.float32) * log2e
        acc = None
        for hh in range(heads):
            s = a_src[:, hh:hh + 1] + a_dst[hh:hh + 1, :]
            s = jnp.maximum(s, 0.2 * s)
            ex = jnp.where(valid, jnp.exp2(s), 0.0)
            o = jax.lax.dot_general(
                ex, ht[:, hh * dim:(hh + 1) * dim],
                (((0,), (0,)), ((), ())),
                preferred_element_type=jnp.float32)
            den = jax.lax.dot_general(
                ex, ones, (((0,), (0,)), ((), ())),
                preferred_element_type=jnp.float32)
            o = o / (den + 1e-16)
            acc = o if acc is None else acc + o
        out_ref[:, t * dim:(t + 1) * dim] = acc * inv_h + b


def kernel(x, node_matrix, W, att_src, att_dst, bias):
    n, t_steps, in_dim = x.shape
    heads, dim = att_src.shape[1], att_src.shape[2]
    hc = heads * dim
    bj = 256
    nj = n // bj

    eye = jnp.eye(heads, dtype=jnp.float32)
    as_bd = (att_src.reshape(heads, dim)[:, :, None]
             * eye[:, None, :]).reshape(hc, heads)
    ad_bd = (att_dst.reshape(heads, dim)[:, None, :]
             * eye[:, :, None]).reshape(heads, hc)
    bias2 = bias.reshape(1, dim).astype(jnp.float32)

    body = functools.partial(_gat_block_kernel, n=n, t_steps=t_steps,
                             heads=heads, dim=dim, bj=bj)
    out = pl.pallas_call(
        body,
        grid=(nj,),
        in_specs=[
            pl.BlockSpec((n, t_steps, in_dim), lambda j: (0, 0, 0)),
            pl.BlockSpec((bj, t_steps, in_dim), lambda j: (j, 0, 0)),
            pl.BlockSpec((in_dim, hc), lambda j: (0, 0)),
            pl.BlockSpec((hc, heads), lambda j: (0, 0)),
            pl.BlockSpec((heads, hc), lambda j: (0, 0)),
            pl.BlockSpec((n, bj), lambda j: (0, j)),
            pl.BlockSpec((1, dim), lambda j: (0, 0)),
        ],
        out_specs=pl.BlockSpec((bj, t_steps * dim), lambda j: (j, 0)),
        out_shape=jax.ShapeDtypeStruct((n, t_steps * dim), jnp.float32),
        compiler_params=pltpu.CompilerParams(
            dimension_semantics=("arbitrary",)),
    )(x.astype(jnp.float32), x.astype(jnp.float32), W, as_bd, ad_bd,
      node_matrix, bias2)
    return out.reshape(n, t_steps, dim)


# speedup vs baseline: 13199.2398x; 1.8062x over previous
import functools

import jax
import jax.numpy as jnp
from jax.experimental import pallas as pl
from jax.experimental.pallas import tpu as pltpu


def _gat_kernel(x_ref, w_ref, as_ref, ad_ref, mask_ref, bias_ref, out_ref,
                *, n, t_steps, heads, dim):
    m = mask_ref[...]
    rows = jax.lax.broadcasted_iota(jnp.int32, (n, n), 0)
    cols = jax.lax.broadcasted_iota(jnp.int32, (n, n), 1)
    maskadd = jnp.where((m != 0) | (rows == cols), 0.0, -1e30)
    w = w_ref[...]
    b = bias_ref[...]
    inv_h = jnp.float32(1.0 / heads)
    log2e = jnp.float32(1.4426950408889634)
    for t in range(t_steps):
        ht = jnp.dot(x_ref[:, t, :], w,
                     preferred_element_type=jnp.float32)
        a_src = jnp.dot(ht, as_ref[...],
                        preferred_element_type=jnp.float32) * log2e
        a_dst = jax.lax.dot_general(
            ad_ref[...], ht, (((1,), (1,)), ((), ())),
            preferred_element_type=jnp.float32) * log2e
        acc = None
        for hh in range(heads):
            s = a_src[:, hh:hh + 1] + a_dst[hh:hh + 1, :] + maskadd
            s = jnp.maximum(s, 0.2 * s)
            ex = jnp.exp2(s)
            o = jax.lax.dot_general(
                ht[:, hh * dim:(hh + 1) * dim], ex,
                (((0,), (0,)), ((), ())),
                preferred_element_type=jnp.float32)
            den = jnp.sum(ex, axis=0, keepdims=True) + 1e-16
            o = o / den
            acc = o if acc is None else acc + o
        out_ref[t * dim:(t + 1) * dim, :] = acc * inv_h + b


def kernel(x, node_matrix, W, att_src, att_dst, bias):
    n, t_steps, in_dim = x.shape
    heads, dim = att_src.shape[1], att_src.shape[2]
    hc = heads * dim

    eye = jnp.eye(heads, dtype=jnp.float32)
    as_bd = (att_src.reshape(heads, dim)[:, :, None]
             * eye[:, None, :]).reshape(hc, heads)
    ad_bd = (att_dst.reshape(heads, dim)[:, None, :]
             * eye[:, :, None]).reshape(heads, hc)
    bias_col = bias.reshape(dim, 1).astype(jnp.float32)

    body = functools.partial(_gat_kernel, n=n, t_steps=t_steps,
                             heads=heads, dim=dim)
    out_t = pl.pallas_call(
        body,
        out_shape=jax.ShapeDtypeStruct((t_steps * dim, n), jnp.float32),
    )(x.astype(jnp.float32), W, as_bd, ad_bd, node_matrix, bias_col)
    return jnp.transpose(out_t.reshape(t_steps, dim, n), (2, 0, 1))


# speedup vs baseline: 13904.9012x; 1.0535x over previous
import functools

import jax
import jax.numpy as jnp
from jax.experimental import pallas as pl
from jax.experimental.pallas import tpu as pltpu


def _gat_kernel(x_ref, w_ref, as_ref, ad_ref, mask_ref, bias_ref, out_ref,
                *, n, t_steps, heads, dim):
    m = mask_ref[...]
    rows = jax.lax.broadcasted_iota(jnp.int32, (n, n), 0)
    cols = jax.lax.broadcasted_iota(jnp.int32, (n, n), 1)
    maskadd = jnp.where((m != 0) | (rows == cols), 0.0, -1e30)
    w = w_ref[...]
    b = bias_ref[...]
    inv_h = jnp.float32(1.0 / heads)
    ones_col = jnp.ones((n, 1), dtype=jnp.bfloat16)
    for t in range(t_steps):
        ht = jnp.dot(x_ref[:, t, :], w,
                     preferred_element_type=jnp.float32)
        a_src = jnp.dot(ht, as_ref[...],
                        preferred_element_type=jnp.float32)
        a_dst = jax.lax.dot_general(
            ad_ref[...], ht, (((1,), (1,)), ((), ())),
            preferred_element_type=jnp.float32)
        ht_bf = ht.astype(jnp.bfloat16)
        acc = None
        for hh in range(heads):
            s = a_src[:, hh:hh + 1] + a_dst[hh:hh + 1, :] + maskadd
            s = jnp.maximum(s, 0.2 * s)
            ex = jnp.exp2(s).astype(jnp.bfloat16)
            lhs = jnp.concatenate(
                [ht_bf[:, hh * dim:(hh + 1) * dim], ones_col], axis=1)
            o_aug = jax.lax.dot_general(
                lhs, ex, (((0,), (0,)), ((), ())),
                preferred_element_type=jnp.float32)
            o = o_aug[:dim, :] / (o_aug[dim:, :] + 1e-16)
            acc = o if acc is None else acc + o
        out_ref[t * dim:(t + 1) * dim, :] = acc * inv_h + b


def kernel(x, node_matrix, W, att_src, att_dst, bias):
    n, t_steps, in_dim = x.shape
    heads, dim = att_src.shape[1], att_src.shape[2]
    hc = heads * dim

    eye = jnp.eye(heads, dtype=jnp.float32)
    log2e = jnp.float32(1.4426950408889634)
    as_bd = (att_src.reshape(heads, dim)[:, :, None]
             * eye[:, None, :]).reshape(hc, heads) * log2e
    ad_bd = (att_dst.reshape(heads, dim)[:, None, :]
             * eye[:, :, None]).reshape(heads, hc) * log2e
    bias_col = bias.reshape(dim, 1).astype(jnp.float32)

    body = functools.partial(_gat_kernel, n=n, t_steps=t_steps,
                             heads=heads, dim=dim)
    out_t = pl.pallas_call(
        body,
        out_shape=jax.ShapeDtypeStruct((t_steps * dim, n), jnp.float32),
    )(x.astype(jnp.float32), W, as_bd, ad_bd, node_matrix, bias_col)
    return jnp.transpose(out_t.reshape(t_steps, dim, n), (2, 0, 1))
